# baseline (speedup 1.0000x reference)
import functools

import jax
import jax.numpy as jnp
from jax import lax
from jax.experimental import pallas as pl
from jax.experimental.pallas import tpu as pltpu
from jax.experimental.pallas import tpu_sc as plsc

_DIM = 64
_ROWS = 40960
_NEG = 5
_BATCH = 4096

_NC = 2
_NS = 16
_NW = _NC * _NS
_G = 16
_RPW = _ROWS // _NW
_CH = _RPW // _G


def _sc_scores(pu, pv, nv, u_weight, v_weight):
  mesh = plsc.VectorSubcoreMesh(core_axis_name="c", subcore_axis_name="s")

  @functools.partial(
      pl.kernel,
      out_type=[
          jax.ShapeDtypeStruct((_ROWS,), jnp.float32),
          jax.ShapeDtypeStruct((_ROWS,), jnp.float32),
      ],
      mesh=mesh,
      compiler_params=pltpu.CompilerParams(
          needs_layout_passes=False, use_tc_tiling_on_sc=False),
      scratch_types=[
          pltpu.VMEM((_CH, 2 * _G), jnp.int32),
          pltpu.VMEM((_CH, 2 * _G), jnp.int32),
          pltpu.VMEM((2 * _CH, 5 * _G), jnp.int32),
          pltpu.VMEM((2 * _G, _DIM), jnp.float32),
          pltpu.VMEM((2 * _G, _DIM), jnp.float32),
          pltpu.VMEM((10 * _G, _DIM), jnp.float32),
          pltpu.VMEM((16, 6 * _G), jnp.float32),
          pltpu.VMEM((_RPW,), jnp.float32),
          pltpu.VMEM((_RPW,), jnp.float32),
          pltpu.SemaphoreType.DMA,
          pltpu.SemaphoreType.DMA,
          pltpu.SemaphoreType.DMA,
          pltpu.SemaphoreType.DMA,
      ],
  )
  def k(pu_hbm, pv_hbm, nv_hbm, u_hbm, v_hbm, score_hbm, negsum_hbm,
        idx_u, idx_v, idx_n, u_rows, v_rows, n_rows, partt, score_all,
        negsum, sem_u, sem_v, sem_n0, sem_n1):
    wid = lax.axis_index("s") * _NC + lax.axis_index("c")
    pltpu.sync_copy(pu_hbm.at[wid], idx_u)
    pltpu.sync_copy(pv_hbm.at[wid], idx_v)
    pltpu.sync_copy(nv_hbm.at[wid], idx_n)

    def chunk(c, carry):
      cu = pltpu.async_copy(u_hbm.at[idx_u.at[c]], u_rows, sem_u)
      cv = pltpu.async_copy(v_hbm.at[idx_v.at[c]], v_rows, sem_v)
      cn0 = pltpu.async_copy(v_hbm.at[idx_n.at[2 * c]],
                             n_rows.at[pl.ds(0, 5 * _G)], sem_n0)
      cn1 = pltpu.async_copy(v_hbm.at[idx_n.at[2 * c + 1]],
                             n_rows.at[pl.ds(5 * _G, 5 * _G)], sem_n1)
      cu.wait()
      cv.wait()
      cn0.wait()
      cn1.wait()
      lanes = lax.iota(jnp.int32, 16)
      for i in range(_G):
        su = [u_rows[2 * i, pl.ds(16 * t, 16)]
              + u_rows[2 * i + 1, pl.ds(16 * t, 16)] for t in range(4)]
        p = su[0] * (v_rows[2 * i, pl.ds(0, 16)]
                     + v_rows[2 * i + 1, pl.ds(0, 16)])
        for t in range(1, 4):
          p = p + su[t] * (v_rows[2 * i, pl.ds(16 * t, 16)]
                           + v_rows[2 * i + 1, pl.ds(16 * t, 16)])
        plsc.store_scatter(partt, [lanes, jnp.full((16,), i, jnp.int32)], p)
        for kk in range(_NEG):
          r = 10 * i + 2 * kk
          pn = su[0] * (n_rows[r, pl.ds(0, 16)] + n_rows[r + 1, pl.ds(0, 16)])
          for t in range(1, 4):
            pn = pn + su[t] * (n_rows[r, pl.ds(16 * t, 16)]
                               + n_rows[r + 1, pl.ds(16 * t, 16)])
          plsc.store_scatter(
              partt, [lanes, jnp.full((16,), (1 + kk) * _G + i, jnp.int32)],
              pn)
      acc = []
      for q in range(1 + _NEG):
        a = partt[0, pl.ds(q * _G, _G)]
        for t in range(1, 16):
          a = a + partt[t, pl.ds(q * _G, _G)]
        acc.append(a)
      ds = pl.ds(c * _G, _G)
      score_all[ds] = acc[0] * 0.25
      s = jnp.exp(acc[1] * 0.25)
      for kk in range(2, 1 + _NEG):
        s = s + jnp.exp(acc[kk] * 0.25)
      negsum[ds] = s
      return carry

    lax.fori_loop(0, _CH, chunk, 0)

    pltpu.sync_copy(score_all, score_hbm.at[pl.ds(wid * _RPW, _RPW)])
    pltpu.sync_copy(negsum, negsum_hbm.at[pl.ds(wid * _RPW, _RPW)])

  return k(pu, pv, nv, u_weight, v_weight)


def _tc_loss(score2d, negsum2d):
  def body(s_ref, n_ref, o_ref):
    val = (jnp.sum(jnp.log(1.0 + n_ref[...]))
           - jnp.sum(s_ref[...])) * (1.0 / _BATCH)
    o_ref[...] = jnp.broadcast_to(val, (1, 1))

  return pl.pallas_call(
      body,
      out_shape=jax.ShapeDtypeStruct((1, 1), jnp.float32),
  )(score2d, negsum2d)


def kernel(pos_u, pos_v, neg_v, u_weight, v_weight):
  pu = pos_u.reshape(_NW, _CH, 2 * _G)
  pv = pos_v.reshape(_NW, _CH, 2 * _G)
  nv = neg_v.reshape(_NW, 2 * _CH, 5 * _G)
  score, negsum = _sc_scores(pu, pv, nv, u_weight, v_weight)
  loss = _tc_loss(score.reshape(_ROWS // 128, 128),
                  negsum.reshape(_ROWS // 128, 128))
  return loss[0, 0]
